# Initial kernel scaffold; baseline (speedup 1.0000x reference)
import jax
import jax.numpy as jnp
from jax.experimental import pallas as pl


def kernel(x, edge_index, W1, b1, W2, b2, Wc, bc):
    raise NotImplementedError("write your pallas kernel here")


# speedup vs baseline: 3.6930x; 3.6930x over previous
import functools

import jax
import jax.numpy as jnp
from jax import lax
from jax.experimental import pallas as pl
from jax.experimental.pallas import tpu as pltpu
from jax.experimental.pallas import tpu_sc as plsc

N = 50000
E = 800000
NB = 98
BK = 512
NPAD = NB * BK
NT = 32
EPT = E // NT
CH = 128
NCHUNK = EPT // CH
TAIL = EPT - NCHUNK * CH
SENT = 127 * BK
E_CAP = E + NB * CH
E_ALL = E_CAP + CH

_MESH = dict(core_axis_name="c", subcore_axis_name="s")


def _wid():
    return lax.axis_index("s") * 2 + lax.axis_index("c")


def _lanes():
    return lax.broadcasted_iota(jnp.int32, (16,), 0)


def _sget(ref, i):
    return plsc.load_gather(ref, [jnp.full((16,), i, jnp.int32)])[0]


def _hist_body(dst_hbm, counts_hbm, dbuf, dbufT, cnt):
    wid = _wid()
    base = wid * EPT
    z16 = jnp.zeros((16,), jnp.int32)
    for g in range(8):
        cnt[pl.ds(g * 16, 16)] = z16

    def count_group(bvec):
        run, last = plsc.scan_count(bvec)
        plsc.addupdate_scatter(cnt, [bvec], run, mask=last)

    def chunk(i, _):
        pltpu.sync_copy(dst_hbm.at[pl.ds(base + i * CH, CH)], dbuf)

        def grp(g, _):
            count_group(lax.shift_right_logical(dbuf[pl.ds(g * 16, 16)], 9))
            return 0

        lax.fori_loop(0, CH // 16, grp, 0)
        return 0

    lax.fori_loop(0, NCHUNK, chunk, 0)

    pltpu.sync_copy(dst_hbm.at[pl.ds(base + NCHUNK * CH, TAIL)],
                    dbufT.at[pl.ds(0, TAIL)])
    for g in range(3):
        dvec = dbufT[pl.ds(g * 16, 16)]
        if (g + 1) * 16 > TAIL:
            dvec = jnp.where(_lanes() < TAIL - g * 16, dvec, SENT)
        count_group(lax.shift_right_logical(dvec, 9))

    pltpu.sync_copy(cnt, counts_hbm.at[wid])


def _sc_hist(dst_e):
    return pl.kernel(
        _hist_body,
        out_type=jax.ShapeDtypeStruct((NT, 128), jnp.int32),
        mesh=plsc.VectorSubcoreMesh(**_MESH),
        compiler_params=pltpu.CompilerParams(needs_layout_passes=False),
        scratch_types=[
            pltpu.VMEM((CH,), jnp.int32),
            pltpu.VMEM((48,), jnp.int32),
            pltpu.VMEM((128,), jnp.int32),
        ],
    )(dst_e)


def _scan_counts(cntall, starts_v, caps_v):
    carry = jnp.int32(0)
    for g in range(8):
        cs = pl.ds(g * 16, 16)

        def acc(t, tot):
            return tot + cntall[t, cs]

        tot = lax.fori_loop(0, NT, acc, jnp.zeros((16,), jnp.int32))
        capc = (tot + (CH - 1)) // CH
        cum = plsc.cumsum(capc)
        starts_v[cs] = cum - capc + carry
        caps_v[cs] = capc
        carry = carry + cum[15]


def _perm_body(src_hbm, dst_hbm, counts_hbm, srcp_hbm, dstp_hbm,
               cntall, sbuf, dbuf, posbuf, sbufT, dbufT, posbufT, cursor,
               gapS_v, gapN_v):
    wid = _wid()
    pltpu.sync_copy(counts_hbm, cntall)

    carry = jnp.int32(0)
    z16 = jnp.zeros((16,), jnp.int32)
    for g in range(8):
        cs = pl.ds(g * 16, 16)

        def acc(t, tm):
            tot, mine = tm
            v = cntall[t, cs]
            return tot + v, mine + jnp.where(t < wid, v, 0)

        tot, mine = lax.fori_loop(0, NT, acc, (z16, z16))
        cap = (tot + (CH - 1)) // CH * CH
        cum = plsc.cumsum(cap)
        sg = cum - cap + carry
        cursor[cs] = sg + mine
        gapS_v[cs] = sg + tot
        gapN_v[cs] = cap - tot
        carry = carry + cum[15]

    def place_group(bvec, out_ref, offset):
        run, last = plsc.scan_count(bvec)
        basev = plsc.load_gather(cursor, [bvec])
        out_ref[pl.ds(offset, 16)] = basev + run - 1
        plsc.addupdate_scatter(cursor, [bvec], run, mask=last)

    base = wid * EPT

    def chunk(i, _):
        off = base + i * CH
        pltpu.sync_copy(src_hbm.at[pl.ds(off, CH)], sbuf)
        pltpu.sync_copy(dst_hbm.at[pl.ds(off, CH)], dbuf)

        def grp(g, _):
            bvec = lax.shift_right_logical(dbuf[pl.ds(g * 16, 16)], 9)
            place_group(bvec, posbuf, g * 16)
            return 0

        lax.fori_loop(0, CH // 16, grp, 0)
        pltpu.sync_copy(sbuf, srcp_hbm.at[posbuf])
        pltpu.sync_copy(dbuf, dstp_hbm.at[posbuf])
        return 0

    lax.fori_loop(0, NCHUNK, chunk, 0)

    off = base + NCHUNK * CH
    pltpu.sync_copy(src_hbm.at[pl.ds(off, TAIL)], sbufT.at[pl.ds(0, TAIL)])
    pltpu.sync_copy(dst_hbm.at[pl.ds(off, TAIL)], dbufT.at[pl.ds(0, TAIL)])
    for g in range(3):
        dvec = dbufT[pl.ds(g * 16, 16)]
        if (g + 1) * 16 > TAIL:
            dvec = jnp.where(_lanes() < TAIL - g * 16, dvec, SENT)
            dbufT[pl.ds(g * 16, 16)] = dvec
        place_group(lax.shift_right_logical(dvec, 9), posbufT, g * 16)
    pltpu.sync_copy(sbufT, srcp_hbm.at[posbufT])
    pltpu.sync_copy(dbufT, dstp_hbm.at[posbufT])

    for k in range(4):
        b = wid + k * NT

        @pl.when(b < NB)
        def _(k=k, b=b):
            gs = _sget(gapS_v, b)
            gn = _sget(gapN_v, b)
            for g in range(8):
                jvec = _lanes() + (g * 16)
                posbuf[pl.ds(g * 16, 16)] = jnp.where(
                    jvec < gn, gs + jvec, E_CAP + jvec
                )
                sbuf[pl.ds(g * 16, 16)] = jnp.zeros((16,), jnp.int32)
                dbuf[pl.ds(g * 16, 16)] = jnp.full((16,), b * BK + BK,
                                                   jnp.int32)
            pltpu.sync_copy(sbuf, srcp_hbm.at[posbuf])
            pltpu.sync_copy(dbuf, dstp_hbm.at[posbuf])


def _sc_permute(src_e, dst_e, counts):
    return pl.kernel(
        _perm_body,
        out_type=(
            jax.ShapeDtypeStruct((E_ALL,), jnp.int32),
            jax.ShapeDtypeStruct((E_ALL,), jnp.int32),
        ),
        mesh=plsc.VectorSubcoreMesh(**_MESH),
        compiler_params=pltpu.CompilerParams(needs_layout_passes=False),
        scratch_types=[
            pltpu.VMEM((NT, 128), jnp.int32),
            pltpu.VMEM((CH,), jnp.int32),
            pltpu.VMEM((CH,), jnp.int32),
            pltpu.VMEM((CH,), jnp.int32),
            pltpu.VMEM((48,), jnp.int32),
            pltpu.VMEM((48,), jnp.int32),
            pltpu.VMEM((48,), jnp.int32),
            pltpu.VMEM((128,), jnp.int32),
            pltpu.VMEM((128,), jnp.int32),
            pltpu.VMEM((128,), jnp.int32),
        ],
    )(src_e, dst_e, counts)


def _deg_body(dstp_hbm, counts_hbm, deg_hbm, cntall, dbuf, starts_v, caps_v,
              slab):
    wid = _wid()
    pltpu.sync_copy(counts_hbm, cntall)
    _scan_counts(cntall, starts_v, caps_v)
    zf = jnp.zeros((16,), jnp.float32)

    for k in range(4):
        b = wid + k * NT

        @pl.when(b < NB)
        def _(b=b):
            def zs(i, _):
                slab[pl.ds(i * 16, 16)] = zf
                return 0

            lax.fori_loop(0, (BK + 32) // 16, zs, 0)
            st = _sget(starts_v, b) * CH
            nch = _sget(caps_v, b)

            def chunk(i, _):
                pltpu.sync_copy(dstp_hbm.at[pl.ds(st + i * CH, CH)], dbuf)

                def grp(g, _):
                    dvec = dbuf[pl.ds(g * 16, 16)] - b * BK
                    run, last = plsc.scan_count(dvec)
                    plsc.addupdate_scatter(slab, [dvec],
                                           run.astype(jnp.float32), mask=last)
                    return 0

                lax.fori_loop(0, CH // 16, grp, 0)
                return 0

            lax.fori_loop(0, nch, chunk, 0)

            def outg(g, _):
                slab[pl.ds(g * 16, 16)] = slab[pl.ds(g * 16, 16)] + 1.0
                return 0

            lax.fori_loop(0, BK // 16, outg, 0)
            pltpu.sync_copy(slab.at[pl.ds(0, BK)],
                            deg_hbm.at[pl.ds(b * BK, BK)])


def _sc_deg(dst_p, counts):
    return pl.kernel(
        _deg_body,
        out_type=jax.ShapeDtypeStruct((NPAD,), jnp.float32),
        mesh=plsc.VectorSubcoreMesh(**_MESH),
        compiler_params=pltpu.CompilerParams(needs_layout_passes=False),
        scratch_types=[
            pltpu.VMEM((NT, 128), jnp.int32),
            pltpu.VMEM((CH,), jnp.int32),
            pltpu.VMEM((128,), jnp.int32),
            pltpu.VMEM((128,), jnp.int32),
            pltpu.VMEM((BK + 32,), jnp.float32),
        ],
    )(dst_p, counts)


def _prop_body(F, srcp_hbm, dstp_hbm, counts_hbm, g_hbm, dinv_hbm, p_hbm,
               cntall, idxbuf, dstbuf, msgbuf, slab, dinvbuf, starts_v,
               caps_v, sem):
    wid = _wid()
    nj = F // 16
    pltpu.sync_copy(counts_hbm, cntall)
    _scan_counts(cntall, starts_v, caps_v)
    zrow = jnp.zeros((16,), jnp.float32)

    for k in range(4):
        b = wid + k * NT

        @pl.when(b < NB)
        def _(b=b):
            def zs(r, _):
                for j in range(nj):
                    slab[r, pl.ds(j * 16, 16)] = zrow
                return 0

            lax.fori_loop(0, BK + 1, zs, 0)

            st = _sget(starts_v, b) * CH
            nch = _sget(caps_v, b)

            def chunk(i, _):
                pltpu.sync_copy(srcp_hbm.at[pl.ds(st + i * CH, CH)], idxbuf)
                pltpu.sync_copy(dstp_hbm.at[pl.ds(st + i * CH, CH)], dstbuf)
                pltpu.async_copy(g_hbm.at[idxbuf], msgbuf, sem).wait()

                def grp(g, _):
                    dvec = dstbuf[pl.ds(g * 16, 16)] - b * BK
                    for l in range(16):
                        d = dvec[l]
                        e = g * 16 + l
                        for j in range(nj):
                            cs = pl.ds(j * 16, 16)
                            plsc.addupdate(slab.at[d, cs], msgbuf[e, cs])
                    return 0

                lax.fori_loop(0, CH // 16, grp, 0)
                return 0

            lax.fori_loop(0, nch, chunk, 0)

            pltpu.sync_copy(dinv_hbm.at[pl.ds(b * BK, BK)], dinvbuf)

            def out_chunk(c, _):
                rows = pl.ds(b * BK + c * CH, CH)
                pltpu.sync_copy(g_hbm.at[rows], msgbuf)

                def rgrp(g, _):
                    dvvec = dinvbuf[pl.ds(c * CH + g * 16, 16)]
                    for l in range(16):
                        r = g * 16 + l
                        dv = jnp.full((16,), dvvec[l], jnp.float32)
                        for j in range(nj):
                            cs = pl.ds(j * 16, 16)
                            lr = c * CH + r
                            msgbuf[r, cs] = (slab[lr, cs] + msgbuf[r, cs]) * dv
                    return 0

                lax.fori_loop(0, CH // 16, rgrp, 0)
                pltpu.sync_copy(msgbuf, p_hbm.at[rows])
                return 0

            lax.fori_loop(0, BK // CH, out_chunk, 0)


def _sc_prop(F, src_p, dst_p, counts, g, dinv):
    return pl.kernel(
        functools.partial(_prop_body, F),
        out_type=jax.ShapeDtypeStruct((NPAD, F), jnp.float32),
        mesh=plsc.VectorSubcoreMesh(**_MESH),
        compiler_params=pltpu.CompilerParams(needs_layout_passes=False),
        scratch_types=[
            pltpu.VMEM((NT, 128), jnp.int32),
            pltpu.VMEM((CH,), jnp.int32),
            pltpu.VMEM((CH,), jnp.int32),
            pltpu.VMEM((CH, F), jnp.float32),
            pltpu.VMEM((BK + 1, F), jnp.float32),
            pltpu.VMEM((BK,), jnp.float32),
            pltpu.VMEM((128,), jnp.int32),
            pltpu.VMEM((128,), jnp.int32),
            pltpu.SemaphoreType.DMA,
        ],
    )(src_p, dst_p, counts, g, dinv)


def _t1_body(deg_ref, x_ref, dinv_ref, g0_ref):
    dv = lax.rsqrt(deg_ref[...])
    dinv_ref[...] = dv
    g0_ref[...] = x_ref[...] * dv


def _tc_stage1(deg2, x_pad):
    return pl.pallas_call(
        _t1_body,
        grid=(NB,),
        in_specs=[
            pl.BlockSpec((BK, 1), lambda i: (i, 0)),
            pl.BlockSpec((BK, 128), lambda i: (i, 0)),
        ],
        out_specs=[
            pl.BlockSpec((BK, 1), lambda i: (i, 0)),
            pl.BlockSpec((BK, 128), lambda i: (i, 0)),
        ],
        out_shape=[
            jax.ShapeDtypeStruct((NPAD, 1), jnp.float32),
            jax.ShapeDtypeStruct((NPAD, 128), jnp.float32),
        ],
    )(deg2, x_pad)


def _t2_body(p_ref, w_ref, b_ref, dinv_ref, g_ref):
    h = jnp.dot(p_ref[...], w_ref[...], preferred_element_type=jnp.float32)
    h = jnp.maximum(h + b_ref[...], 0.0)
    g_ref[...] = h * dinv_ref[...]


def _tc_layer(p, w, bvec, dinv2):
    fin = p.shape[1]
    return pl.pallas_call(
        _t2_body,
        grid=(NB,),
        in_specs=[
            pl.BlockSpec((BK, fin), lambda i: (i, 0)),
            pl.BlockSpec((fin, 128), lambda i: (0, 0)),
            pl.BlockSpec((1, 128), lambda i: (0, 0)),
            pl.BlockSpec((BK, 1), lambda i: (i, 0)),
        ],
        out_specs=pl.BlockSpec((BK, 128), lambda i: (i, 0)),
        out_shape=jax.ShapeDtypeStruct((NPAD, 128), jnp.float32),
    )(p, w, bvec, dinv2)


def _t4_body(p_ref, w2_ref, b2_ref, wc_ref, bc_ref, out_ref):
    h = jnp.dot(p_ref[...], w2_ref[...], preferred_element_type=jnp.float32)
    h = jnp.maximum(h + b2_ref[...], 0.0)
    out_ref[...] = (
        jnp.dot(h, wc_ref[...], preferred_element_type=jnp.float32)
        + bc_ref[...]
    )


def _tc_final(p2, w2, b2v, wcp, bcp):
    return pl.pallas_call(
        _t4_body,
        grid=(NB,),
        in_specs=[
            pl.BlockSpec((BK, 128), lambda i: (i, 0)),
            pl.BlockSpec((128, 128), lambda i: (0, 0)),
            pl.BlockSpec((1, 128), lambda i: (0, 0)),
            pl.BlockSpec((128, 8), lambda i: (0, 0)),
            pl.BlockSpec((1, 8), lambda i: (0, 0)),
        ],
        out_specs=pl.BlockSpec((BK, 8), lambda i: (i, 0)),
        out_shape=jax.ShapeDtypeStruct((NPAD, 8), jnp.float32),
    )(p2, w2, b2v, wcp, bcp)


def kernel(x, edge_index, W1, b1, W2, b2, Wc, bc):
    src_e = edge_index[0]
    dst_e = edge_index[1]
    x_pad = jnp.pad(x, ((0, NPAD - N), (0, 128 - x.shape[1])))
    W1p = jnp.pad(W1, ((0, 128 - W1.shape[0]), (0, 0)))
    Wcp = jnp.pad(Wc, ((0, 0), (0, 8 - Wc.shape[1])))
    b1r = b1.reshape(1, 128)
    b2r = b2.reshape(1, 128)
    bcp = jnp.pad(bc, (0, 8 - bc.shape[0])).reshape(1, 8)

    counts = _sc_hist(dst_e)
    src_p, dst_p = _sc_permute(src_e, dst_e, counts)
    deg = _sc_deg(dst_p, counts)
    dinv2, g0 = _tc_stage1(deg.reshape(NPAD, 1), x_pad)
    dinv = dinv2.reshape(NPAD)

    p0 = _sc_prop(128, src_p, dst_p, counts, g0, dinv)
    g1 = _tc_layer(p0, W1p, b1r, dinv2)
    p1 = _sc_prop(128, src_p, dst_p, counts, g1, dinv)
    g2 = _tc_layer(p1, W2, b2r, dinv2)
    p2 = _sc_prop(128, src_p, dst_p, counts, g2, dinv)
    out = _tc_final(p2, W2, b2r, Wcp, bcp)
    return out[:N, :2]


# speedup vs baseline: 6.8671x; 1.8595x over previous
import jax
import jax.numpy as jnp
from jax import lax
from jax.experimental import pallas as pl
from jax.experimental.pallas import tpu as pltpu
from jax.experimental.pallas import tpu_sc as plsc

N = 50000
E = 800000
NB = 98
BK = 512
NPAD = NB * BK
NT = 32
EPT = E // NT
CH = 128
SUP = 5
NSUP = EPT // (CH * SUP)
TAIL = EPT - NSUP * CH * SUP
SENTB = 127
E_CAP = E + NB * CH
E_ALL = E_CAP + CH
MLOW = 0xFFFF
MKEEP = 0x01FFFFFF

_MESH = dict(core_axis_name="c", subcore_axis_name="s")
_CP = dict(compiler_params=pltpu.CompilerParams(needs_layout_passes=False))


def _wid():
    return lax.axis_index("s") * 2 + lax.axis_index("c")


def _lanes():
    return lax.broadcasted_iota(jnp.int32, (16,), 0)


def _sget(ref, i):
    return plsc.load_gather(ref, [jnp.full((16,), i, jnp.int32)])[0]


def _srl(x, n):
    return lax.shift_right_logical(x, jnp.full(x.shape, n, jnp.int32))


def _hist_body(src_hbm, dst_hbm, counts_hbm, packed_hbm, sbuf, dbuf, pbuf,
               cnt):
    wid = _wid()
    base = wid * EPT
    z16 = jnp.zeros((16,), jnp.int32)
    for g in range(8):
        cnt[pl.ds(g * 16, 16)] = z16

    def count_group(bvec):
        run, last = plsc.scan_count(bvec)
        plsc.addupdate_scatter(cnt, [bvec], run, mask=last)

    def chunk(i, _):
        off = base + i * CH
        pltpu.sync_copy(src_hbm.at[pl.ds(off, CH)], sbuf)
        pltpu.sync_copy(dst_hbm.at[pl.ds(off, CH)], dbuf)

        def grp(g, _):
            cs = pl.ds(g * 16, 16)
            dv = dbuf[cs]
            count_group(_srl(dv, 9))
            pbuf[cs] = sbuf[cs] | lax.shift_left(dv, 16)
            return 0

        lax.fori_loop(0, CH // 16, grp, 0)
        pltpu.sync_copy(pbuf, packed_hbm.at[pl.ds(off, CH)])
        return 0

    lax.fori_loop(0, EPT // CH, chunk, 0)

    off = base + (EPT // CH) * CH
    pltpu.sync_copy(src_hbm.at[pl.ds(off, TAIL)], sbuf.at[pl.ds(0, TAIL)])
    pltpu.sync_copy(dst_hbm.at[pl.ds(off, TAIL)], dbuf.at[pl.ds(0, TAIL)])
    for g in range(3):
        cs = pl.ds(g * 16, 16)
        dv = dbuf[cs]
        bvec = _srl(dv, 9)
        if (g + 1) * 16 > TAIL:
            bvec = jnp.where(_lanes() < TAIL - g * 16, bvec, SENTB)
        count_group(bvec)
        pbuf[cs] = sbuf[cs] | lax.shift_left(dv, 16)
    pltpu.sync_copy(pbuf.at[pl.ds(0, TAIL)],
                    packed_hbm.at[pl.ds(off, TAIL)])

    pltpu.sync_copy(cnt, counts_hbm.at[wid])


def _sc_hist(src_e, dst_e):
    return pl.kernel(
        _hist_body,
        out_type=(
            jax.ShapeDtypeStruct((NT, 128), jnp.int32),
            jax.ShapeDtypeStruct((E,), jnp.int32),
        ),
        mesh=plsc.VectorSubcoreMesh(**_MESH),
        **_CP,
        scratch_types=[
            pltpu.VMEM((CH,), jnp.int32),
            pltpu.VMEM((CH,), jnp.int32),
            pltpu.VMEM((CH,), jnp.int32),
            pltpu.VMEM((128,), jnp.int32),
        ],
    )(src_e, dst_e)


def _scan_counts(cntall, starts_v, caps_v):
    carry = jnp.int32(0)
    for g in range(8):
        cs = pl.ds(g * 16, 16)

        def acc(t, tot):
            return tot + cntall[t, cs]

        tot = lax.fori_loop(0, NT, acc, jnp.zeros((16,), jnp.int32))
        capc = (tot + (CH - 1)) // CH
        cum = plsc.cumsum(capc)
        starts_v[cs] = cum - capc + carry
        caps_v[cs] = capc
        carry = carry + cum[15]


def _perm_body(packed_hbm, counts_hbm, packedp_hbm,
               cntall, ebig, posbuf2, posbufT, cursor, gapS_v, gapN_v, sem):
    wid = _wid()
    pltpu.sync_copy(counts_hbm, cntall)

    carry = jnp.int32(0)
    z16 = jnp.zeros((16,), jnp.int32)
    for g in range(8):
        cs = pl.ds(g * 16, 16)

        def acc(t, tm):
            tot, mine = tm
            v = cntall[t, cs]
            return tot + v, mine + jnp.where(t < wid, v, 0)

        tot, mine = lax.fori_loop(0, NT, acc, (z16, z16))
        cap = (tot + (CH - 1)) // CH * CH
        cum = plsc.cumsum(cap)
        sg = cum - cap + carry
        cursor[cs] = sg + mine
        gapS_v[cs] = sg + tot
        gapN_v[cs] = cap - tot
        carry = carry + cum[15]

    def place_group(bvec, out_ref, offset):
        run, last = plsc.scan_count(bvec)
        basev = plsc.load_gather(cursor, [bvec])
        out_ref[pl.ds(offset, 16)] = basev + run - 1
        plsc.addupdate_scatter(cursor, [bvec], run, mask=last)

    base = wid * EPT

    def superchunk(s, _):
        off = base + s * (CH * SUP)
        pltpu.sync_copy(packed_hbm.at[pl.ds(off, CH * SUP)], ebig)
        for j in range(SUP):
            def grp(g, _, j=j):
                cs = pl.ds(j * CH + g * 16, 16)
                ev = ebig[cs]
                place_group(_srl(ev, 25), posbuf2.at[j], g * 16)
                ebig[cs] = ev & MKEEP
                return 0

            lax.fori_loop(0, CH // 16, grp, 0)
            pltpu.async_copy(ebig.at[pl.ds(j * CH, CH)],
                             packedp_hbm.at[posbuf2.at[j]], sem)
        for j in range(SUP):
            pltpu.make_async_copy(ebig.at[pl.ds(j * CH, CH)],
                                  packedp_hbm.at[posbuf2.at[j]], sem).wait()
        return 0

    lax.fori_loop(0, NSUP, superchunk, 0)

    off = base + NSUP * CH * SUP
    pltpu.sync_copy(packed_hbm.at[pl.ds(off, TAIL)], ebig.at[pl.ds(0, TAIL)])
    for g in range(3):
        cs = pl.ds(g * 16, 16)
        ev = ebig[cs]
        bvec = _srl(ev, 25)
        if (g + 1) * 16 > TAIL:
            bvec = jnp.where(_lanes() < TAIL - g * 16, bvec, SENTB)
        place_group(bvec, posbufT, g * 16)
        ebig[cs] = ev & MKEEP
    pltpu.sync_copy(ebig.at[pl.ds(0, 48)], packedp_hbm.at[posbufT])

    for k in range(4):
        b = wid + k * NT

        @pl.when(b < NB)
        def _(k=k, b=b):
            gs = _sget(gapS_v, b)
            gn = _sget(gapN_v, b)
            for g in range(8):
                jvec = _lanes() + (g * 16)
                posbuf2[0, pl.ds(g * 16, 16)] = jnp.where(
                    jvec < gn, gs + jvec, E_CAP + jvec
                )
                ebig[pl.ds(g * 16, 16)] = jnp.full((16,), BK << 16, jnp.int32)
            pltpu.sync_copy(ebig.at[pl.ds(0, CH)],
                            packedp_hbm.at[posbuf2.at[0]])


def _sc_permute(packed_e, counts):
    return pl.kernel(
        _perm_body,
        out_type=jax.ShapeDtypeStruct((E_ALL,), jnp.int32),
        mesh=plsc.VectorSubcoreMesh(**_MESH),
        **_CP,
        scratch_types=[
            pltpu.VMEM((NT, 128), jnp.int32),
            pltpu.VMEM((CH * SUP,), jnp.int32),
            pltpu.VMEM((SUP, CH), jnp.int32),
            pltpu.VMEM((48,), jnp.int32),
            pltpu.VMEM((128,), jnp.int32),
            pltpu.VMEM((128,), jnp.int32),
            pltpu.VMEM((128,), jnp.int32),
            pltpu.SemaphoreType.DMA,
        ],
    )(packed_e, counts)


def _deg_body(packedp_hbm, counts_hbm, deg_hbm, cntall, dbuf, starts_v,
              caps_v, slab):
    wid = _wid()
    pltpu.sync_copy(counts_hbm, cntall)
    _scan_counts(cntall, starts_v, caps_v)
    zf = jnp.zeros((16,), jnp.float32)

    for k in range(4):
        b = wid + k * NT

        @pl.when(b < NB)
        def _(b=b):
            def zs(i, _):
                slab[pl.ds(i * 16, 16)] = zf
                return 0

            lax.fori_loop(0, (BK + 32) // 16, zs, 0)
            st = _sget(starts_v, b) * CH
            nch = _sget(caps_v, b)

            def chunk(i, _):
                pltpu.sync_copy(packedp_hbm.at[pl.ds(st + i * CH, CH)], dbuf)

                def grp(g, _):
                    dvec = _srl(dbuf[pl.ds(g * 16, 16)], 16)
                    run, last = plsc.scan_count(dvec)
                    plsc.addupdate_scatter(slab, [dvec],
                                           run.astype(jnp.float32), mask=last)
                    return 0

                lax.fori_loop(0, CH // 16, grp, 0)
                return 0

            lax.fori_loop(0, nch, chunk, 0)

            def outg(g, _):
                slab[pl.ds(g * 16, 16)] = slab[pl.ds(g * 16, 16)] + 1.0
                return 0

            lax.fori_loop(0, BK // 16, outg, 0)
            pltpu.sync_copy(slab.at[pl.ds(0, BK)],
                            deg_hbm.at[pl.ds(b * BK, BK)])


def _sc_deg(packed_p, counts):
    return pl.kernel(
        _deg_body,
        out_type=jax.ShapeDtypeStruct((NPAD,), jnp.float32),
        mesh=plsc.VectorSubcoreMesh(**_MESH),
        **_CP,
        scratch_types=[
            pltpu.VMEM((NT, 128), jnp.int32),
            pltpu.VMEM((CH,), jnp.int32),
            pltpu.VMEM((128,), jnp.int32),
            pltpu.VMEM((128,), jnp.int32),
            pltpu.VMEM((BK + 32,), jnp.float32),
        ],
    )(packed_p, counts)


def _prop_body(packedp_hbm, counts_hbm, g_hbm, dinv_hbm, p_hbm,
               cntall, ebufA, ebufB, idxA, idxB, msgA, msgB, slab, dinvbuf,
               starts_v, caps_v, semA, semB, semW):
    wid = _wid()
    pltpu.sync_copy(counts_hbm, cntall)
    _scan_counts(cntall, starts_v, caps_v)
    zrow = jnp.zeros((16,), jnp.float32)

    def issue(st, c, eb, ib, mb, sm):
        pltpu.sync_copy(packedp_hbm.at[pl.ds(st + c * CH, CH)], eb)
        for g in range(8):
            cs = pl.ds(g * 16, 16)
            ib[cs] = eb[cs] & MLOW
        pltpu.async_copy(g_hbm.at[ib], mb, sm)

    def drain(ib, mb, sm):
        pltpu.make_async_copy(g_hbm.at[ib], mb, sm).wait()

    def accum(eb, mb):
        def grp(g, _):
            dvec = _srl(eb[pl.ds(g * 16, 16)], 16)
            for l0 in range(0, 16, 2):
                d0 = dvec[l0]
                d1 = dvec[l0 + 1]
                e0 = g * 16 + l0
                v0 = [mb[e0, pl.ds(j * 16, 16)] for j in range(8)]
                v1 = [mb[e0 + 1, pl.ds(j * 16, 16)] for j in range(8)]
                for j in range(8):
                    plsc.addupdate(slab.at[d0, pl.ds(j * 16, 16)], v0[j])
                for j in range(8):
                    plsc.addupdate(slab.at[d1, pl.ds(j * 16, 16)], v1[j])
            return 0

        lax.fori_loop(0, CH // 16, grp, 0)

    def bucket(k, _):
        b = wid + k * NT

        @pl.when(b < NB)
        def _():
            def zs(r, _):
                for j in range(8):
                    slab[r, pl.ds(j * 16, 16)] = zrow
                return 0

            lax.fori_loop(0, BK + 1, zs, 0)

            st = _sget(starts_v, b) * CH
            nch = _sget(caps_v, b)

            @pl.when(nch > 0)
            def _():
                issue(st, 0, ebufA, idxA, msgA, semA)

                def pair(ip, _):
                    c0 = ip * 2

                    @pl.when(c0 + 1 < nch)
                    def _():
                        issue(st, c0 + 1, ebufB, idxB, msgB, semB)

                    drain(idxA, msgA, semA)
                    accum(ebufA, msgA)

                    @pl.when(c0 + 2 < nch)
                    def _():
                        issue(st, c0 + 2, ebufA, idxA, msgA, semA)

                    @pl.when(c0 + 1 < nch)
                    def _():
                        drain(idxB, msgB, semB)
                        accum(ebufB, msgB)

                    return 0

                lax.fori_loop(0, (nch + 1) // 2, pair, 0)

            pltpu.sync_copy(dinv_hbm.at[pl.ds(b * BK, BK)], dinvbuf)
            for c in range(4):
                buf = msgA if c % 2 == 0 else msgB
                if c >= 2:
                    prows = pl.ds(b * BK + (c - 2) * CH, CH)
                    pltpu.make_async_copy(buf, p_hbm.at[prows], semW).wait()

                def rgrp(g, _, c=c, buf=buf):
                    dvvec = dinvbuf[pl.ds(c * CH + g * 16, 16)]
                    for l in range(16):
                        r = g * 16 + l
                        lr = c * CH + r
                        dv = jnp.full((16,), dvvec[l], jnp.float32)
                        sv = [slab[lr, pl.ds(j * 16, 16)] for j in range(8)]
                        for j in range(8):
                            buf[r, pl.ds(j * 16, 16)] = sv[j] * dv
                    return 0

                lax.fori_loop(0, CH // 16, rgrp, 0)
                rows = pl.ds(b * BK + c * CH, CH)
                pltpu.async_copy(buf, p_hbm.at[rows], semW)
            for c in range(2, 4):
                buf = msgA if c % 2 == 0 else msgB
                rows = pl.ds(b * BK + c * CH, CH)
                pltpu.make_async_copy(buf, p_hbm.at[rows], semW).wait()

        return 0

    lax.fori_loop(0, 4, bucket, 0)


def _sc_prop(packed_p, counts, g, dinv):
    return pl.kernel(
        _prop_body,
        out_type=jax.ShapeDtypeStruct((NPAD, 128), jnp.float32),
        mesh=plsc.VectorSubcoreMesh(**_MESH),
        **_CP,
        scratch_types=[
            pltpu.VMEM((NT, 128), jnp.int32),
            pltpu.VMEM((CH,), jnp.int32),
            pltpu.VMEM((CH,), jnp.int32),
            pltpu.VMEM((CH,), jnp.int32),
            pltpu.VMEM((CH,), jnp.int32),
            pltpu.VMEM((CH, 128), jnp.float32),
            pltpu.VMEM((CH, 128), jnp.float32),
            pltpu.VMEM((BK + 1, 128), jnp.float32),
            pltpu.VMEM((BK,), jnp.float32),
            pltpu.VMEM((128,), jnp.int32),
            pltpu.VMEM((128,), jnp.int32),
            pltpu.SemaphoreType.DMA,
            pltpu.SemaphoreType.DMA,
            pltpu.SemaphoreType.DMA,
        ],
    )(packed_p, counts, g, dinv)


def _t1_body(deg_ref, x_ref, dinv_ref, g0_ref, q0_ref):
    dv = lax.rsqrt(deg_ref[...])
    g0 = x_ref[...] * dv
    dinv_ref[...] = dv
    g0_ref[...] = g0
    q0_ref[...] = g0 * dv


def _tc_stage1(deg2, x_pad):
    return pl.pallas_call(
        _t1_body,
        grid=(NB,),
        in_specs=[
            pl.BlockSpec((BK, 1), lambda i: (i, 0)),
            pl.BlockSpec((BK, 128), lambda i: (i, 0)),
        ],
        out_specs=[
            pl.BlockSpec((BK, 1), lambda i: (i, 0)),
            pl.BlockSpec((BK, 128), lambda i: (i, 0)),
            pl.BlockSpec((BK, 128), lambda i: (i, 0)),
        ],
        out_shape=[
            jax.ShapeDtypeStruct((NPAD, 1), jnp.float32),
            jax.ShapeDtypeStruct((NPAD, 128), jnp.float32),
            jax.ShapeDtypeStruct((NPAD, 128), jnp.float32),
        ],
    )(deg2, x_pad)


def _t2_body(ps_ref, q_ref, w_ref, b_ref, dinv_ref, g_ref, qo_ref):
    p = ps_ref[...] + q_ref[...]
    h = jnp.dot(p, w_ref[...], preferred_element_type=jnp.float32)
    h = jnp.maximum(h + b_ref[...], 0.0)
    dv = dinv_ref[...]
    g = h * dv
    g_ref[...] = g
    qo_ref[...] = g * dv


def _tc_layer(ps, q, w, bvec, dinv2):
    return pl.pallas_call(
        _t2_body,
        grid=(NB,),
        in_specs=[
            pl.BlockSpec((BK, 128), lambda i: (i, 0)),
            pl.BlockSpec((BK, 128), lambda i: (i, 0)),
            pl.BlockSpec((128, 128), lambda i: (0, 0)),
            pl.BlockSpec((1, 128), lambda i: (0, 0)),
            pl.BlockSpec((BK, 1), lambda i: (i, 0)),
        ],
        out_specs=[
            pl.BlockSpec((BK, 128), lambda i: (i, 0)),
            pl.BlockSpec((BK, 128), lambda i: (i, 0)),
        ],
        out_shape=[
            jax.ShapeDtypeStruct((NPAD, 128), jnp.float32),
            jax.ShapeDtypeStruct((NPAD, 128), jnp.float32),
        ],
    )(ps, q, w, bvec, dinv2)


def _t4_body(ps_ref, q_ref, w2_ref, b2_ref, wc_ref, bc_ref, out_ref):
    p = ps_ref[...] + q_ref[...]
    h = jnp.dot(p, w2_ref[...], preferred_element_type=jnp.float32)
    h = jnp.maximum(h + b2_ref[...], 0.0)
    out_ref[...] = (
        jnp.dot(h, wc_ref[...], preferred_element_type=jnp.float32)
        + bc_ref[...]
    )


def _tc_final(ps2, q2, w2, b2v, wcp, bcp):
    return pl.pallas_call(
        _t4_body,
        grid=(NB,),
        in_specs=[
            pl.BlockSpec((BK, 128), lambda i: (i, 0)),
            pl.BlockSpec((BK, 128), lambda i: (i, 0)),
            pl.BlockSpec((128, 128), lambda i: (0, 0)),
            pl.BlockSpec((1, 128), lambda i: (0, 0)),
            pl.BlockSpec((128, 8), lambda i: (0, 0)),
            pl.BlockSpec((1, 8), lambda i: (0, 0)),
        ],
        out_specs=pl.BlockSpec((BK, 8), lambda i: (i, 0)),
        out_shape=jax.ShapeDtypeStruct((NPAD, 8), jnp.float32),
    )(ps2, q2, w2, b2v, wcp, bcp)


def kernel(x, edge_index, W1, b1, W2, b2, Wc, bc):
    src_e = edge_index[0]
    dst_e = edge_index[1]
    x_pad = jnp.pad(x, ((0, NPAD - N), (0, 128 - x.shape[1])))
    W1p = jnp.pad(W1, ((0, 128 - W1.shape[0]), (0, 0)))
    Wcp = jnp.pad(Wc, ((0, 0), (0, 8 - Wc.shape[1])))
    b1r = b1.reshape(1, 128)
    b2r = b2.reshape(1, 128)
    bcp = jnp.pad(bc, (0, 8 - bc.shape[0])).reshape(1, 8)

    counts, packed_e = _sc_hist(src_e, dst_e)
    packed_p = _sc_permute(packed_e, counts)
    deg = _sc_deg(packed_p, counts)
    dinv2, g0, q0 = _tc_stage1(deg.reshape(NPAD, 1), x_pad)
    dinv = dinv2.reshape(NPAD)

    ps0 = _sc_prop(packed_p, counts, g0, dinv)
    g1, q1 = _tc_layer(ps0, q0, W1p, b1r, dinv2)
    ps1 = _sc_prop(packed_p, counts, g1, dinv)
    g2, q2 = _tc_layer(ps1, q1, W2, b2r, dinv2)
    ps2 = _sc_prop(packed_p, counts, g2, dinv)
    out = _tc_final(ps2, q2, W2, b2r, Wcp, bcp)
    return out[:N, :2]
